# baseline (speedup 1.0000x reference)
import functools
import jax
import jax.numpy as jnp
from jax.experimental import pallas as pl
from jax.experimental.pallas import tpu as pltpu

_BM = 128
_BI = 2048


def _router_body(x_ref, wr_ref, eid_ref, prob_ref, *, n_exp):
    logits = jnp.dot(x_ref[...], wr_ref[...], preferred_element_type=jnp.float32)
    col = jax.lax.broadcasted_iota(jnp.int32, logits.shape, 1)
    valid = col < n_exp
    neg = jnp.where(valid, logits, -1e30)
    m = jnp.max(neg, axis=1, keepdims=True)
    p = jnp.exp(neg - m)
    s = jnp.sum(p, axis=1, keepdims=True)
    probs = p / s
    pmax = jnp.max(probs, axis=1, keepdims=True)
    is_max = (probs == pmax) & valid
    eid_ref[...] = jnp.min(jnp.where(is_max, col, n_exp), axis=1, keepdims=True)
    prob_ref[...] = pmax


def _mlp_body(te_ref, tv_ref, x_ref, w1_ref, w2_ref, pr_ref, out_ref):
    t = pl.program_id(1)

    @pl.when(tv_ref[t] == 1)
    def _compute():
        h1 = jnp.dot(x_ref[...], w1_ref[0], preferred_element_type=jnp.float32)
        act = 0.5 * h1 * (1.0 + jax.lax.erf(h1 * 0.7071067811865476))
        act = act * pr_ref[:, :1]
        out_ref[0] = jnp.dot(act, w2_ref[0], preferred_element_type=jnp.float32)

    @pl.when(tv_ref[t] == 0)
    def _zero():
        out_ref[...] = jnp.zeros_like(out_ref)


def kernel(hidden_states, Wr, W1, W2):
    B, S, H = hidden_states.shape
    E, _, I = W1.shape
    T = B * S
    hs = hidden_states.reshape(T, H)

    wr_pad = jnp.zeros((H, 128), jnp.float32).at[:, :E].set(Wr)
    eid2, prob2 = pl.pallas_call(
        functools.partial(_router_body, n_exp=E),
        out_shape=[
            jax.ShapeDtypeStruct((T, 1), jnp.int32),
            jax.ShapeDtypeStruct((T, 1), jnp.float32),
        ],
    )(hs, wr_pad)
    eid = eid2[:, 0]
    prob = prob2[:, 0]

    NT = T // _BM + E
    P = NT * _BM
    onehot = (eid[:, None] == jnp.arange(E)[None, :]).astype(jnp.int32)
    counts = onehot.sum(axis=0)
    rank = (jnp.cumsum(onehot, axis=0) - onehot)
    rank = (rank * onehot).sum(axis=1)
    padded = ((counts + _BM - 1) // _BM) * _BM
    ends = jnp.cumsum(padded)
    offsets = ends - padded
    pos = offsets[eid] + rank
    perm = jnp.zeros((P,), jnp.int32).at[pos].set(jnp.arange(T, dtype=jnp.int32))
    row0 = jnp.minimum(jnp.arange(NT, dtype=jnp.int32) * _BM, ends[-1] - 1)
    tile_expert = jnp.minimum(
        jnp.searchsorted(ends, row0, side="right").astype(jnp.int32), E - 1)
    tile_valid = (jnp.arange(NT, dtype=jnp.int32) * _BM < ends[-1]).astype(jnp.int32)

    x_sorted = jnp.take(hs, perm, axis=0)
    prob_sorted = jnp.take(prob, perm, axis=0)
    prob_b = jnp.broadcast_to(prob_sorted[:, None], (P, 128))

    NI = I // _BI
    grid_spec = pltpu.PrefetchScalarGridSpec(
        num_scalar_prefetch=2,
        grid=(NI, NT),
        in_specs=[
            pl.BlockSpec((_BM, H), lambda i, t, te, tv: (t, 0)),
            pl.BlockSpec((1, H, _BI), lambda i, t, te, tv: (te[t], 0, i)),
            pl.BlockSpec((1, _BI, H), lambda i, t, te, tv: (te[t], i, 0)),
            pl.BlockSpec((_BM, 128), lambda i, t, te, tv: (t, 0)),
        ],
        out_specs=pl.BlockSpec((1, _BM, H), lambda i, t, te, tv: (i, t, 0)),
    )
    out_parts = pl.pallas_call(
        _mlp_body,
        grid_spec=grid_spec,
        out_shape=jax.ShapeDtypeStruct((NI, P, H), jnp.float32),
    )(tile_expert, tile_valid, x_sorted, W1, W2, prob_b)
    out_sorted = out_parts.sum(axis=0)

    out = jnp.take(out_sorted, pos, axis=0)
    return out.reshape(B, S, H)


# speedup vs baseline: 1.0859x; 1.0859x over previous
import functools
import jax
import jax.numpy as jnp
from jax.experimental import pallas as pl
from jax.experimental.pallas import tpu as pltpu

_BM = 128
_BI = 2048


def _scan_incl(x, axis):
    n = x.shape[axis]
    k = 1
    while k < n:
        zshape = list(x.shape)
        zshape[axis] = k
        sl = [slice(None)] * x.ndim
        sl[axis] = slice(0, n - k)
        x = x + jnp.concatenate(
            [jnp.zeros(zshape, x.dtype), x[tuple(sl)]], axis=axis)
        k *= 2
    return x


def _plan_body(x_ref, wr_ref, pos_ref, perm_ref, prob_ref, te_ref, tv_ref,
               *, n_exp, n_tiles, n_slots):
    T = x_ref.shape[0]
    logits = jnp.dot(x_ref[...], wr_ref[...], preferred_element_type=jnp.float32)
    col = jax.lax.broadcasted_iota(jnp.int32, logits.shape, 1)
    lane_ok = col < n_exp
    neg = jnp.where(lane_ok, logits, -1e30)
    m = jnp.max(neg, axis=1, keepdims=True)
    p = jnp.exp(neg - m)
    probs = p / jnp.sum(p, axis=1, keepdims=True)
    pmax = jnp.max(probs, axis=1, keepdims=True)
    is_max = (probs == pmax) & lane_ok
    eid = jnp.min(jnp.where(is_max, col, n_exp), axis=1, keepdims=True)
    onehot = (col == eid).astype(jnp.float32)

    csum = _scan_incl(onehot, axis=0)
    rank = jnp.sum((csum - onehot) * onehot, axis=1, keepdims=True)
    counts = jnp.sum(onehot, axis=0, keepdims=True)
    padded = jnp.floor((counts + (_BM - 1)) * (1.0 / _BM)) * _BM
    ends = _scan_incl(jnp.where(lane_ok[:1], padded, 0.0), axis=1)
    offsets = ends - padded
    total = jnp.max(ends, axis=1, keepdims=True)

    pos_f = rank + jnp.sum(onehot * offsets, axis=1, keepdims=True)
    pos_ref[...] = pos_f.astype(jnp.int32)

    trow = jax.lax.broadcasted_iota(jnp.int32, (n_tiles, 128), 0).astype(jnp.float32) * _BM
    row0 = jnp.minimum(trow, total - 1.0)
    ends_b = jnp.broadcast_to(ends, (n_tiles, 128))
    lane2 = jax.lax.broadcasted_iota(jnp.int32, (n_tiles, 128), 1)
    te = jnp.sum(jnp.where((ends_b <= row0) & (lane2 < n_exp), 1, 0),
                 axis=1, keepdims=True)
    te_ref[...] = jnp.minimum(te, n_exp - 1)
    tv_ref[...] = (trow[:, :1] < total).astype(jnp.int32)

    slot = jax.lax.broadcasted_iota(jnp.int32, (T, n_slots), 1).astype(jnp.float32)
    eq = (pos_f == slot).astype(jnp.float32)
    t_f = jax.lax.broadcasted_iota(jnp.int32, (T, 128), 0).astype(jnp.float32)
    a_d = jnp.floor(t_f * (1.0 / 128.0))
    b_d = t_f - a_d * 128.0
    p_hi = jnp.broadcast_to(pmax, (T, 128)).astype(jnp.bfloat16).astype(jnp.float32)
    p_lo = jnp.broadcast_to(pmax, (T, 128)) - p_hi
    rhs = jnp.where(col == 0, a_d,
          jnp.where(col == 1, b_d,
          jnp.where(col == 2, p_hi,
          jnp.where(col == 3, p_lo, 0.0))))
    dn = (((0,), (0,)), ((), ()))
    r = jax.lax.dot_general(eq, rhs, dn, preferred_element_type=jnp.float32)
    perm_ref[...] = (r[:, 0:1] * 128.0 + r[:, 1:2]).astype(jnp.int32)
    prob_ref[...] = jnp.broadcast_to(r[:, 2:3] + r[:, 3:4], (n_slots, 128))


def _mlp_body(te_ref, tv_ref, x_ref, w1_ref, w2_ref, pr_ref, out_ref):
    t = pl.program_id(1)

    @pl.when(tv_ref[t] == 1)
    def _compute():
        h1 = jnp.dot(x_ref[...], w1_ref[0], preferred_element_type=jnp.float32)
        act = 0.5 * h1 * (1.0 + jax.lax.erf(h1 * 0.7071067811865476))
        act = act * pr_ref[:, :1]
        out_ref[0] = jnp.dot(act, w2_ref[0], preferred_element_type=jnp.float32)

    @pl.when(tv_ref[t] == 0)
    def _zero():
        out_ref[...] = jnp.zeros_like(out_ref)


def kernel(hidden_states, Wr, W1, W2):
    B, S, H = hidden_states.shape
    E, _, I = W1.shape
    T = B * S
    hs = hidden_states.reshape(T, H)
    NT = T // _BM + E
    P = NT * _BM

    wr_pad = jnp.zeros((H, 128), jnp.float32).at[:, :E].set(Wr)
    pos2, perm2, prob_b, te2, tv2 = pl.pallas_call(
        functools.partial(_plan_body, n_exp=E, n_tiles=NT, n_slots=P),
        out_shape=[
            jax.ShapeDtypeStruct((T, 1), jnp.int32),
            jax.ShapeDtypeStruct((P, 1), jnp.int32),
            jax.ShapeDtypeStruct((P, 128), jnp.float32),
            jax.ShapeDtypeStruct((NT, 1), jnp.int32),
            jax.ShapeDtypeStruct((NT, 1), jnp.int32),
        ],
    )(hs, wr_pad)
    pos = pos2[:, 0]

    x_sorted = jnp.take(hs, perm2[:, 0], axis=0)

    NI = I // _BI
    grid_spec = pltpu.PrefetchScalarGridSpec(
        num_scalar_prefetch=2,
        grid=(NI, NT),
        in_specs=[
            pl.BlockSpec((_BM, H), lambda i, t, te, tv: (t, 0)),
            pl.BlockSpec((1, H, _BI), lambda i, t, te, tv: (te[t], 0, i)),
            pl.BlockSpec((1, _BI, H), lambda i, t, te, tv: (te[t], i, 0)),
            pl.BlockSpec((_BM, 128), lambda i, t, te, tv: (t, 0)),
        ],
        out_specs=pl.BlockSpec((1, _BM, H), lambda i, t, te, tv: (i, t, 0)),
    )
    out_parts = pl.pallas_call(
        _mlp_body,
        grid_spec=grid_spec,
        out_shape=jax.ShapeDtypeStruct((NI, P, H), jnp.float32),
    )(te2[:, 0], tv2[:, 0], x_sorted, W1, W2, prob_b)
    out_sorted = out_parts.sum(axis=0)

    out = jnp.take(out_sorted, pos, axis=0)
    return out.reshape(B, S, H)


# speedup vs baseline: 1.1304x; 1.0410x over previous
import functools
import jax
import jax.numpy as jnp
from jax import lax
from jax.experimental import pallas as pl
from jax.experimental.pallas import tpu as pltpu
from jax.experimental.pallas import tpu_sc as plsc

_SC_NC, _SC_NS = 2, 16
_SC_NW = _SC_NC * _SC_NS

_BM = 128
_BI = 2048


def _scan_incl(x, axis):
    n = x.shape[axis]
    k = 1
    while k < n:
        zshape = list(x.shape)
        zshape[axis] = k
        sl = [slice(None)] * x.ndim
        sl[axis] = slice(0, n - k)
        x = x + jnp.concatenate(
            [jnp.zeros(zshape, x.dtype), x[tuple(sl)]], axis=axis)
        k *= 2
    return x


def _plan_body(x_ref, wr_ref, pos_ref, perm_ref, prob_ref, te_ref, tv_ref,
               *, n_exp, n_tiles, n_slots):
    T = x_ref.shape[0]
    logits = jnp.dot(x_ref[...], wr_ref[...], preferred_element_type=jnp.float32)
    col = jax.lax.broadcasted_iota(jnp.int32, logits.shape, 1)
    lane_ok = col < n_exp
    neg = jnp.where(lane_ok, logits, -1e30)
    m = jnp.max(neg, axis=1, keepdims=True)
    p = jnp.exp(neg - m)
    probs = p / jnp.sum(p, axis=1, keepdims=True)
    pmax = jnp.max(probs, axis=1, keepdims=True)
    is_max = (probs == pmax) & lane_ok
    eid = jnp.min(jnp.where(is_max, col, n_exp), axis=1, keepdims=True)
    onehot = (col == eid).astype(jnp.float32)

    csum = _scan_incl(onehot, axis=0)
    rank = jnp.sum((csum - onehot) * onehot, axis=1, keepdims=True)
    counts = jnp.sum(onehot, axis=0, keepdims=True)
    padded = jnp.floor((counts + (_BM - 1)) * (1.0 / _BM)) * _BM
    ends = _scan_incl(jnp.where(lane_ok[:1], padded, 0.0), axis=1)
    offsets = ends - padded
    total = jnp.max(ends, axis=1, keepdims=True)

    pos_f = rank + jnp.sum(onehot * offsets, axis=1, keepdims=True)
    pos_ref[...] = pos_f.astype(jnp.int32)

    trow = jax.lax.broadcasted_iota(jnp.int32, (n_tiles, 128), 0).astype(jnp.float32) * _BM
    row0 = jnp.minimum(trow, total - 1.0)
    ends_b = jnp.broadcast_to(ends, (n_tiles, 128))
    lane2 = jax.lax.broadcasted_iota(jnp.int32, (n_tiles, 128), 1)
    te = jnp.sum(jnp.where((ends_b <= row0) & (lane2 < n_exp), 1, 0),
                 axis=1, keepdims=True)
    te_ref[...] = jnp.minimum(te, n_exp - 1)
    tv_ref[...] = (trow[:, :1] < total).astype(jnp.int32)

    slot = jax.lax.broadcasted_iota(jnp.int32, (T, n_slots), 1).astype(jnp.float32)
    eq = (pos_f == slot).astype(jnp.float32)
    t_f = jax.lax.broadcasted_iota(jnp.int32, (T, 128), 0).astype(jnp.float32)
    a_d = jnp.floor(t_f * (1.0 / 128.0))
    b_d = t_f - a_d * 128.0
    p_hi = jnp.broadcast_to(pmax, (T, 128)).astype(jnp.bfloat16).astype(jnp.float32)
    p_lo = jnp.broadcast_to(pmax, (T, 128)) - p_hi
    rhs = jnp.where(col == 0, a_d,
          jnp.where(col == 1, b_d,
          jnp.where(col == 2, p_hi,
          jnp.where(col == 3, p_lo, 0.0))))
    dn = (((0,), (0,)), ((), ()))
    r = jax.lax.dot_general(eq, rhs, dn, preferred_element_type=jnp.float32)
    perm_ref[...] = (r[:, 0:1] * 128.0 + r[:, 1:2]).astype(jnp.int32)
    prob_ref[...] = jnp.broadcast_to(r[:, 2:3] + r[:, 3:4], (n_slots, 128))


def _mlp_body(te_ref, tv_ref, x_ref, w1_ref, w2_ref, pr_ref, out_ref):
    t = pl.program_id(1)

    @pl.when(tv_ref[t] == 1)
    def _compute():
        h1 = jnp.dot(x_ref[...], w1_ref[0], preferred_element_type=jnp.float32)
        act = 0.5 * h1 * (1.0 + jax.lax.erf(h1 * 0.7071067811865476))
        act = act * pr_ref[:, :1]
        out_ref[0] = jnp.dot(act, w2_ref[0], preferred_element_type=jnp.float32)

    @pl.when(tv_ref[t] == 0)
    def _zero():
        out_ref[...] = jnp.zeros_like(out_ref)


def kernel(hidden_states, Wr, W1, W2):
    B, S, H = hidden_states.shape
    E, _, I = W1.shape
    T = B * S
    hs = hidden_states.reshape(T, H)
    NT = T // _BM + E
    P = NT * _BM

    wr_pad = jnp.zeros((H, 128), jnp.float32).at[:, :E].set(Wr)
    pos2, perm2, prob_b, te2, tv2 = pl.pallas_call(
        functools.partial(_plan_body, n_exp=E, n_tiles=NT, n_slots=P),
        out_shape=[
            jax.ShapeDtypeStruct((T, 1), jnp.int32),
            jax.ShapeDtypeStruct((P, 1), jnp.int32),
            jax.ShapeDtypeStruct((P, 128), jnp.float32),
            jax.ShapeDtypeStruct((NT, 1), jnp.int32),
            jax.ShapeDtypeStruct((NT, 1), jnp.int32),
        ],
    )(hs, wr_pad)
    pos = pos2[:, 0]

    b_per_w = P // _SC_NW

    def _dispatch_body(table_hbm, idx_hbm, out_hbm, idx_v, rows_v, sem):
        wid = lax.axis_index("s") * _SC_NC + lax.axis_index("c")
        base = wid * b_per_w
        pltpu.sync_copy(idx_hbm.at[pl.ds(base, b_per_w)], idx_v)
        pltpu.async_copy(table_hbm.at[idx_v], rows_v, sem).wait()
        pltpu.sync_copy(rows_v, out_hbm.at[pl.ds(base, b_per_w)])

    x_sorted = pl.kernel(
        _dispatch_body,
        out_type=jax.ShapeDtypeStruct((P, H), jnp.float32),
        mesh=plsc.VectorSubcoreMesh(core_axis_name="c", subcore_axis_name="s"),
        scratch_types=[
            pltpu.VMEM((b_per_w,), jnp.int32),
            pltpu.VMEM((b_per_w, H), jnp.float32),
            pltpu.SemaphoreType.DMA,
        ],
    )(hs, perm2[:, 0])

    NI = I // _BI
    grid_spec = pltpu.PrefetchScalarGridSpec(
        num_scalar_prefetch=2,
        grid=(NI, NT),
        in_specs=[
            pl.BlockSpec((_BM, H), lambda i, t, te, tv: (t, 0)),
            pl.BlockSpec((1, H, _BI), lambda i, t, te, tv: (te[t], 0, i)),
            pl.BlockSpec((1, _BI, H), lambda i, t, te, tv: (te[t], i, 0)),
            pl.BlockSpec((_BM, 128), lambda i, t, te, tv: (t, 0)),
        ],
        out_specs=pl.BlockSpec((1, _BM, H), lambda i, t, te, tv: (i, t, 0)),
    )
    out_parts = pl.pallas_call(
        _mlp_body,
        grid_spec=grid_spec,
        out_shape=jax.ShapeDtypeStruct((NI, P, H), jnp.float32),
    )(te2[:, 0], tv2[:, 0], x_sorted, W1, W2, prob_b)
    out_sorted = out_parts.sum(axis=0)

    out = jnp.take(out_sorted, pos, axis=0)
    return out.reshape(B, S, H)


# speedup vs baseline: 1.1570x; 1.0235x over previous
import functools
import jax
import jax.numpy as jnp
from jax import lax
from jax.experimental import pallas as pl
from jax.experimental.pallas import tpu as pltpu
from jax.experimental.pallas import tpu_sc as plsc

_SC_NC, _SC_NS = 2, 16
_SC_NW = _SC_NC * _SC_NS

_BM = 128
_BI = 2048


def _scan_incl(x, axis):
    n = x.shape[axis]
    k = 1
    while k < n:
        zshape = list(x.shape)
        zshape[axis] = k
        sl = [slice(None)] * x.ndim
        sl[axis] = slice(0, n - k)
        x = x + jnp.concatenate(
            [jnp.zeros(zshape, x.dtype), x[tuple(sl)]], axis=axis)
        k *= 2
    return x


def _plan_body(x_ref, wr_ref, pos_ref, perm_ref, prob_ref, te_ref, tv_ref,
               *, n_exp, n_tiles, n_slots):
    T = x_ref.shape[0]
    logits = jnp.dot(x_ref[...], wr_ref[...], preferred_element_type=jnp.float32)
    col = jax.lax.broadcasted_iota(jnp.int32, logits.shape, 1)
    lane_ok = col < n_exp
    neg = jnp.where(lane_ok, logits, -1e30)
    m = jnp.max(neg, axis=1, keepdims=True)
    p = jnp.exp(neg - m)
    probs = p / jnp.sum(p, axis=1, keepdims=True)
    pmax = jnp.max(probs, axis=1, keepdims=True)
    is_max = (probs == pmax) & lane_ok
    eid = jnp.min(jnp.where(is_max, col, n_exp), axis=1, keepdims=True)
    onehot = (col == eid).astype(jnp.float32)

    csum = _scan_incl(onehot, axis=0)
    rank = jnp.sum((csum - onehot) * onehot, axis=1, keepdims=True)
    counts = jnp.sum(onehot, axis=0, keepdims=True)
    padded = jnp.floor((counts + (_BM - 1)) * (1.0 / _BM)) * _BM
    ends = _scan_incl(jnp.where(lane_ok[:1], padded, 0.0), axis=1)
    offsets = ends - padded
    total = jnp.max(ends, axis=1, keepdims=True)

    pos_f = rank + jnp.sum(onehot * offsets, axis=1, keepdims=True)
    pos_ref[...] = pos_f.astype(jnp.int32)

    trow = jax.lax.broadcasted_iota(jnp.int32, (n_tiles, 128), 0).astype(jnp.float32) * _BM
    row0 = jnp.minimum(trow, total - 1.0)
    ends_b = jnp.broadcast_to(ends, (n_tiles, 128))
    lane2 = jax.lax.broadcasted_iota(jnp.int32, (n_tiles, 128), 1)
    te = jnp.sum(jnp.where((ends_b <= row0) & (lane2 < n_exp), 1, 0),
                 axis=1, keepdims=True)
    te_ref[...] = jnp.minimum(te, n_exp - 1)
    tv_ref[...] = (trow[:, :1] < total).astype(jnp.int32)

    slot = jax.lax.broadcasted_iota(jnp.int32, (T, n_slots), 1).astype(jnp.float32)
    eq = (pos_f == slot).astype(jnp.float32)
    t_f = jax.lax.broadcasted_iota(jnp.int32, (T, 128), 0).astype(jnp.float32)
    a_d = jnp.floor(t_f * (1.0 / 128.0))
    b_d = t_f - a_d * 128.0
    p_hi = jnp.broadcast_to(pmax, (T, 128)).astype(jnp.bfloat16).astype(jnp.float32)
    p_lo = jnp.broadcast_to(pmax, (T, 128)) - p_hi
    rhs = jnp.where(col == 0, a_d,
          jnp.where(col == 1, b_d,
          jnp.where(col == 2, p_hi,
          jnp.where(col == 3, p_lo, 0.0))))
    dn = (((0,), (0,)), ((), ()))
    r = jax.lax.dot_general(eq, rhs, dn, preferred_element_type=jnp.float32)
    perm_ref[...] = (r[:, 0:1] * 128.0 + r[:, 1:2]).astype(jnp.int32)
    prob_ref[...] = jnp.broadcast_to(r[:, 2:3] + r[:, 3:4], (n_slots, 128))


def _sc_row_gather(table, idx, n_out):
    n_rows, width = table.shape
    b_per_w = n_out // _SC_NW

    def _body(table_hbm, idx_hbm, out_hbm, idx_v, rows_v, sem):
        wid = lax.axis_index("s") * _SC_NC + lax.axis_index("c")
        base = wid * b_per_w
        pltpu.sync_copy(idx_hbm.at[pl.ds(base, b_per_w)], idx_v)
        pltpu.async_copy(table_hbm.at[idx_v], rows_v, sem).wait()
        pltpu.sync_copy(rows_v, out_hbm.at[pl.ds(base, b_per_w)])

    return pl.kernel(
        _body,
        out_type=jax.ShapeDtypeStruct((n_out, width), table.dtype),
        mesh=plsc.VectorSubcoreMesh(core_axis_name="c", subcore_axis_name="s"),
        scratch_types=[
            pltpu.VMEM((b_per_w,), jnp.int32),
            pltpu.VMEM((b_per_w, width), table.dtype),
            pltpu.SemaphoreType.DMA,
        ],
    )(table, idx)


def _mlp_body(te_ref, tv_ref, x_ref, w1_ref, w2_ref, pr_ref, out_ref):
    t = pl.program_id(1)

    @pl.when(tv_ref[t] == 1)
    def _compute():
        h1 = jnp.dot(x_ref[...], w1_ref[0], preferred_element_type=jnp.float32)
        act = 0.5 * h1 * (1.0 + jax.lax.erf(h1 * 0.7071067811865476))
        act = act * pr_ref[:, :1]
        out_ref[0] = jnp.dot(act, w2_ref[0], preferred_element_type=jnp.float32)

    @pl.when(tv_ref[t] == 0)
    def _zero():
        out_ref[...] = jnp.zeros_like(out_ref)


def kernel(hidden_states, Wr, W1, W2):
    B, S, H = hidden_states.shape
    E, _, I = W1.shape
    T = B * S
    hs = hidden_states.reshape(T, H)
    NT = T // _BM + E
    P = NT * _BM

    wr_pad = jnp.zeros((H, 128), jnp.float32).at[:, :E].set(Wr)
    pos2, perm2, prob_b, te2, tv2 = pl.pallas_call(
        functools.partial(_plan_body, n_exp=E, n_tiles=NT, n_slots=P),
        out_shape=[
            jax.ShapeDtypeStruct((T, 1), jnp.int32),
            jax.ShapeDtypeStruct((P, 1), jnp.int32),
            jax.ShapeDtypeStruct((P, 128), jnp.float32),
            jax.ShapeDtypeStruct((NT, 1), jnp.int32),
            jax.ShapeDtypeStruct((NT, 1), jnp.int32),
        ],
    )(hs, wr_pad)
    pos = pos2[:, 0]

    x_sorted = _sc_row_gather(hs, perm2[:, 0], P)

    NI = I // _BI
    grid_spec = pltpu.PrefetchScalarGridSpec(
        num_scalar_prefetch=2,
        grid=(NI, NT),
        in_specs=[
            pl.BlockSpec((_BM, H), lambda i, t, te, tv: (t, 0)),
            pl.BlockSpec((1, H, _BI), lambda i, t, te, tv: (te[t], 0, i)),
            pl.BlockSpec((1, _BI, H), lambda i, t, te, tv: (te[t], i, 0)),
            pl.BlockSpec((_BM, 128), lambda i, t, te, tv: (t, 0)),
        ],
        out_specs=pl.BlockSpec((1, _BM, H), lambda i, t, te, tv: (i, t, 0)),
    )
    out_parts = pl.pallas_call(
        _mlp_body,
        grid_spec=grid_spec,
        out_shape=jax.ShapeDtypeStruct((NI, P, H), jnp.float32),
    )(te2[:, 0], tv2[:, 0], x_sorted, W1, W2, prob_b)
    out_sorted = out_parts.sum(axis=0)

    out = _sc_row_gather(out_sorted, pos, T)
    return out.reshape(B, S, H)
